# baseline (speedup 1.0000x reference)
import functools
import math

import jax
import jax.numpy as jnp
from jax import lax
from jax.experimental import pallas as pl
from jax.experimental.pallas import tpu as pltpu
from jax.experimental.pallas import tpu_sc as plsc

_B, _C, _T, _H, _W = 16, 768, 8, 12, 30
_N = _H * _W
_K = 120
_R = 2 + _K + _N
_NW = 32
_CHUNK = 64
_SCROWS = _R - 128
_WROWS = _SCROWS // 2


def _scores_kernel(x_hbm, cls_ref, p_ref, s0, sem0, sem1, sem2, sem3):
    i = pl.program_id(0)
    sems = (sem0, sem1, sem2, sem3)
    cps = [
        pltpu.make_async_copy(x_hbm.at[i, pl.ds(96 * j, 96 if j < 3 else 72),
                                       0, :],
                              s0.at[pl.ds(96 * j, 96 if j < 3 else 72), :],
                              sems[j])
        for j in range(4)
    ]
    for cp in cps:
        cp.start()
    for cp in cps:
        cp.wait()
    x0t = s0[...]
    cls0 = cls_ref[0, 0:1, :]
    s = jax.lax.dot_general(
        cls0, x0t, (((1,), (1,)), ((), ())),
        preferred_element_type=jnp.float32) / math.sqrt(_C)
    p_ref[0] = jax.nn.softmax(s, axis=-1)


def _local_kernel(x_hbm, cls_ref, p_all_ref, p_mine_ref, glob_ref, out_ref,
                  s0, s6, sem0, sem1, sem2, sem3, sem6):
    i = pl.program_id(0)
    sems = (sem0, sem1, sem2, sem3)
    cps = [
        pltpu.make_async_copy(x_hbm.at[i, pl.ds(96 * j, 96 if j < 3 else 72),
                                       0, :],
                              s0.at[pl.ds(96 * j, 96 if j < 3 else 72), :],
                              sems[j])
        for j in range(4)
    ]
    cp6 = pltpu.make_async_copy(x_hbm.at[i, 0:8, 1, :], s6, sem6)
    for cp in cps:
        cp.start()
    cp6.start()

    norm = jnp.max(p_all_ref[...])
    q = p_mine_ref[0] / norm
    qT = jnp.transpose(q)

    row = jax.lax.broadcasted_iota(jnp.int32, (_N, _N), 0)
    col = jax.lax.broadcasted_iota(jnp.int32, (_N, _N), 1)
    cmp = (qT > q) | ((qT == q) & (row < col))
    rank = jnp.sum(cmp.astype(jnp.int32), axis=0, keepdims=True)

    k_iota = jax.lax.broadcasted_iota(jnp.int32, (_K, _N), 0)
    sel = (k_iota == rank).astype(jnp.float32)

    for cp in cps:
        cp.wait()
    local = jax.lax.dot_general(
        sel, s0[...], (((1,), (0,)), ((), ())),
        precision=jax.lax.Precision.HIGHEST,
        preferred_element_type=jnp.float32)

    out_ref[0, 0:1, :] = cls_ref[0, 0:1, :]
    out_ref[0, 1:1 + _K, :] = local
    out_ref[0, 1 + _K:2 + _K, :] = cls_ref[0, 1:2, :]
    cp6.wait()
    out_ref[0, 2 + _K:, :] = s6[0:128 - 2 - _K, :]


def _make_glob():
    mesh = plsc.VectorSubcoreMesh(core_axis_name="c", subcore_axis_name="s")

    @functools.partial(
        pl.kernel,
        mesh=mesh,
        out_type=jax.ShapeDtypeStruct((_B, _R, _C), jnp.float32),
        scratch_types=[
            pltpu.VMEM((3, _CHUNK), jnp.int32),
            pltpu.VMEM((3, _CHUNK), jnp.int32),
            pltpu.VMEM((_CHUNK, _C), jnp.float32),
            pltpu.SemaphoreType.DMA,
            pltpu.SemaphoreType.DMA,
        ],
    )
    def _glob(xflat_hbm, src_hbm, dst_hbm, out_hbm,
              src_v, dst_v, rows_v, gsem, ssem):
        cid = lax.axis_index("c")
        sid = lax.axis_index("s")
        w = sid * 2 + cid
        pltpu.sync_copy(src_hbm.at[w], src_v)
        pltpu.sync_copy(dst_hbm.at[w], dst_v)
        for j in range(3):
            pltpu.async_copy(xflat_hbm.at[src_v.at[j]], rows_v, gsem).wait()
            pltpu.async_copy(rows_v, out_hbm.at[sid].at[dst_v.at[j]],
                             ssem).wait()

    return _glob


def kernel(x, cls_tokens):
    b, c, t, h, w = x.shape
    n = h * w
    xt4 = jnp.transpose(x, (0, 3, 4, 2, 1)).reshape(b, n, t, c)
    xflat = xt4.reshape(b * n * t, c)

    j_idx = jnp.minimum(
        jnp.arange(3, dtype=jnp.int32)[:, None] * _CHUNK
        + jnp.arange(_CHUNK, dtype=jnp.int32)[None, :],
        _WROWS - 1)
    half = (jnp.arange(_NW, dtype=jnp.int32) % 2)[:, None, None]
    batch = (jnp.arange(_NW, dtype=jnp.int32) // 2)[:, None, None]
    dst_map = 128 + half * _WROWS + j_idx[None]
    tok = dst_map - (2 + _K)
    src_map = batch * (n * t) + tok * t + 1

    p = pl.pallas_call(
        _scores_kernel,
        grid=(b,),
        in_specs=[
            pl.BlockSpec(memory_space=pl.ANY),
            pl.BlockSpec((1, t, c), lambda i: (i, 0, 0)),
        ],
        out_specs=pl.BlockSpec((1, 1, n), lambda i: (i, 0, 0)),
        out_shape=jax.ShapeDtypeStruct((b, 1, n), jnp.float32),
        scratch_shapes=[
            pltpu.VMEM((n, c), jnp.float32),
            pltpu.SemaphoreType.DMA,
            pltpu.SemaphoreType.DMA,
            pltpu.SemaphoreType.DMA,
            pltpu.SemaphoreType.DMA,
        ],
    )(xt4, cls_tokens)

    glob_out = _make_glob()(xflat, src_map, dst_map)

    out = pl.pallas_call(
        _local_kernel,
        grid=(b,),
        in_specs=[
            pl.BlockSpec(memory_space=pl.ANY),
            pl.BlockSpec((1, t, c), lambda i: (i, 0, 0)),
            pl.BlockSpec((b, 1, n), lambda i: (0, 0, 0)),
            pl.BlockSpec((1, 1, n), lambda i: (i, 0, 0)),
            pl.BlockSpec(memory_space=pl.ANY),
        ],
        out_specs=pl.BlockSpec((1, 128, c), lambda i: (i, 0, 0)),
        out_shape=jax.ShapeDtypeStruct((b, _R, c), jnp.float32),
        input_output_aliases={4: 0},
        scratch_shapes=[
            pltpu.VMEM((n, c), jnp.float32),
            pltpu.VMEM((8, c), jnp.float32),
            pltpu.SemaphoreType.DMA,
            pltpu.SemaphoreType.DMA,
            pltpu.SemaphoreType.DMA,
            pltpu.SemaphoreType.DMA,
            pltpu.SemaphoreType.DMA,
        ],
    )(xt4, cls_tokens, p, p, glob_out)
    return out


# speedup vs baseline: 1.1979x; 1.1979x over previous
import functools
import math

import jax
import jax.numpy as jnp
from jax import lax
from jax.experimental import pallas as pl
from jax.experimental.pallas import tpu as pltpu
from jax.experimental.pallas import tpu_sc as plsc

_B, _C, _T, _H, _W = 16, 768, 8, 12, 30
_N = _H * _W
_K = 120
_R = 2 + _K + _N
_HALF = _R // 2
_NW = 32
_CHUNK = 64
_TB = 4


def _scores_kernel(x_hbm, cls_ref, p_ref, s0, sem):
    i = pl.program_id(0)
    nb = pl.num_programs(0)
    slot = lax.rem(i, 2)
    nxt = lax.rem(i + 1, 2)

    @pl.when(i == 0)
    def _():
        pltpu.make_async_copy(x_hbm.at[0, :, 0, :], s0.at[0], sem.at[0]).start()

    @pl.when(i + 1 < nb)
    def _():
        pltpu.make_async_copy(x_hbm.at[i + 1, :, 0, :], s0.at[nxt],
                              sem.at[nxt]).start()

    pltpu.make_async_copy(x_hbm.at[i, :, 0, :], s0.at[slot],
                          sem.at[slot]).wait()
    x0t = s0[slot]
    cls0 = cls_ref[0, 0:1, :]
    s = jax.lax.dot_general(
        cls0, x0t, (((1,), (1,)), ((), ())),
        preferred_element_type=jnp.float32) / math.sqrt(_C)
    p_ref[0] = jax.nn.softmax(s, axis=-1)


def _topk_kernel(pfull_ref, prow_ref, pcol_ref, idx_ref):
    norm = jnp.max(pfull_ref[...])
    q_row = prow_ref[...] / norm
    q_col = pcol_ref[...] / norm
    m_iota = jax.lax.broadcasted_iota(jnp.int32, (_TB, _N, _N), 1)
    n_iota = jax.lax.broadcasted_iota(jnp.int32, (_TB, _N, _N), 2)
    cmp = (q_col > q_row) | ((q_col == q_row) & (m_iota < n_iota))
    rank = jnp.sum(cmp.astype(jnp.int32), axis=1, keepdims=True)
    k_iota = jax.lax.broadcasted_iota(jnp.int32, (_TB, _K, _N), 1)
    t_iota = jax.lax.broadcasted_iota(jnp.int32, (_TB, _K, _N), 2)
    onehot = (k_iota == rank).astype(jnp.int32)
    ids = jnp.sum(onehot * t_iota, axis=2, keepdims=True)
    idx_ref[:, 0:_K, :] = ids
    idx_ref[:, _K:, :] = jnp.zeros((_TB, 128 - _K, 1), jnp.int32)


def _make_assemble():
    mesh = plsc.VectorSubcoreMesh(core_axis_name="c", subcore_axis_name="s")

    @functools.partial(
        pl.kernel,
        mesh=mesh,
        out_type=jax.ShapeDtypeStruct((_B, _R, _C), jnp.float32),
        scratch_types=[
            pltpu.VMEM((4, _CHUNK), jnp.int32),
            pltpu.VMEM((4, _CHUNK), jnp.int32),
            pltpu.VMEM((_CHUNK, _C), jnp.float32),
            pltpu.SemaphoreType.DMA,
            pltpu.SemaphoreType.DMA,
        ],
    )
    def _assemble(xflat_hbm, cls_hbm, src_hbm, dst_hbm, out_hbm,
                  src_v, dst_v, rows_v, gsem, ssem):
        cid = lax.axis_index("c")
        sid = lax.axis_index("s")
        w = sid * 2 + cid
        pltpu.sync_copy(src_hbm.at[w], src_v)
        pltpu.sync_copy(dst_hbm.at[w], dst_v)
        for j in range(4):
            pltpu.async_copy(xflat_hbm.at[src_v.at[j]], rows_v, gsem).wait()
            if j == 0:
                @pl.when(cid == 0)
                def _():
                    pltpu.sync_copy(cls_hbm.at[sid * 8], rows_v.at[0])
            if j == 1:
                @pl.when(cid == 0)
                def _():
                    pltpu.sync_copy(cls_hbm.at[sid * 8 + 1], rows_v.at[57])
            pltpu.async_copy(rows_v, out_hbm.at[sid].at[dst_v.at[j]],
                             ssem).wait()

    return _assemble


def kernel(x, cls_tokens):
    b, c, t, h, w = x.shape
    n = h * w
    xt4 = jnp.transpose(x, (0, 3, 4, 2, 1)).reshape(b, n, t, c)
    xflat = xt4.reshape(b * n * t, c)
    cls_flat = cls_tokens.reshape(b * t, c)

    p = pl.pallas_call(
        _scores_kernel,
        grid=(b,),
        in_specs=[
            pl.BlockSpec(memory_space=pl.ANY),
            pl.BlockSpec((1, t, c), lambda i: (i, 0, 0)),
        ],
        out_specs=pl.BlockSpec((1, 1, n), lambda i: (i, 0, 0)),
        out_shape=jax.ShapeDtypeStruct((b, 1, n), jnp.float32),
        scratch_shapes=[
            pltpu.VMEM((2, n, c), jnp.float32),
            pltpu.SemaphoreType.DMA((2,)),
        ],
    )(xt4, cls_tokens)

    p_col = p.reshape(b, n, 1)
    sel = pl.pallas_call(
        _topk_kernel,
        grid=(b // _TB,),
        in_specs=[
            pl.BlockSpec((b, 1, n), lambda i: (0, 0, 0)),
            pl.BlockSpec((_TB, 1, n), lambda i: (i, 0, 0)),
            pl.BlockSpec((_TB, n, 1), lambda i: (i, 0, 0)),
        ],
        out_specs=pl.BlockSpec((_TB, 128, 1), lambda i: (i, 0, 0)),
        out_shape=jax.ShapeDtypeStruct((b, 128, 1), jnp.int32),
    )(p, p, p_col)
    sel_ids = sel[:, :_K, 0]

    batch_base = (jnp.arange(b, dtype=jnp.int32) * (n * t))[:, None]
    sel_rows = batch_base + sel_ids * t
    glob_rows = batch_base + jnp.arange(n, dtype=jnp.int32)[None, :] * t + 1
    zero = jnp.zeros((b, 1), jnp.int32)
    row_map = jnp.concatenate(
        [batch_base + zero, sel_rows, batch_base + zero, glob_rows], axis=1)
    row_map = row_map.reshape(_NW, _HALF)

    j_idx = jnp.minimum(
        jnp.arange(4, dtype=jnp.int32)[:, None] * _CHUNK
        + jnp.arange(_CHUNK, dtype=jnp.int32)[None, :],
        _HALF - 1)
    src_map = jnp.take_along_axis(
        row_map[:, None, :], j_idx[None], axis=2)
    dst_map = ((jnp.arange(_NW, dtype=jnp.int32) % 2) * _HALF)[:, None, None] \
        + j_idx[None]

    return _make_assemble()(xflat, cls_flat, src_map, dst_map)


# speedup vs baseline: 1.2081x; 1.0085x over previous
import functools
import math

import jax
import jax.numpy as jnp
from jax import lax
from jax.experimental import pallas as pl
from jax.experimental.pallas import tpu as pltpu
from jax.experimental.pallas import tpu_sc as plsc

_B, _C, _T, _H, _W = 16, 768, 8, 12, 30
_N = _H * _W
_K = 120
_R = 2 + _K + _N
_HALF = _R // 2
_NW = 32
_CHUNK = 64
_TB = 4


def _scores_kernel(x_hbm, cls_ref, p_ref, s0, sem):
    i = pl.program_id(0)
    nb = pl.num_programs(0)
    slot = lax.rem(i, 2)
    nxt = lax.rem(i + 1, 2)

    @pl.when(i == 0)
    def _():
        pltpu.make_async_copy(x_hbm.at[0, :, 0, :], s0.at[0], sem.at[0]).start()

    @pl.when(i + 1 < nb)
    def _():
        pltpu.make_async_copy(x_hbm.at[i + 1, :, 0, :], s0.at[nxt],
                              sem.at[nxt]).start()

    pltpu.make_async_copy(x_hbm.at[i, :, 0, :], s0.at[slot],
                          sem.at[slot]).wait()
    x0t = s0[slot]
    cls0 = cls_ref[0, 0:1, :]
    s = jax.lax.dot_general(
        cls0, x0t, (((1,), (1,)), ((), ())),
        preferred_element_type=jnp.float32) / math.sqrt(_C)
    p_ref[0] = jax.nn.softmax(s, axis=-1)


def _topk_kernel(pfull_ref, prow_ref, pcol_ref, idx_ref):
    norm = jnp.max(pfull_ref[...])
    q_row = prow_ref[...] / norm
    q_col = pcol_ref[...] / norm
    m_iota = jax.lax.broadcasted_iota(jnp.int32, (_TB, _N, _N), 1)
    n_iota = jax.lax.broadcasted_iota(jnp.int32, (_TB, _N, _N), 2)
    cmp = (q_col > q_row) | ((q_col == q_row) & (m_iota < n_iota))
    rank = jnp.sum(cmp.astype(jnp.int32), axis=1, keepdims=True)
    k_iota = jax.lax.broadcasted_iota(jnp.int32, (_TB, _K, _N), 1)
    t_iota = jax.lax.broadcasted_iota(jnp.int32, (_TB, _K, _N), 2)
    onehot = (k_iota == rank).astype(jnp.int32)
    ids = jnp.sum(onehot * t_iota, axis=2, keepdims=True)
    idx_ref[:, 0:_K, :] = ids
    idx_ref[:, _K:, :] = jnp.zeros((_TB, 128 - _K, 1), jnp.int32)


def _make_assemble():
    mesh = plsc.VectorSubcoreMesh(core_axis_name="c", subcore_axis_name="s")

    @functools.partial(
        pl.kernel,
        mesh=mesh,
        out_type=jax.ShapeDtypeStruct((_B, _R, _C), jnp.float32),
        scratch_types=[
            pltpu.VMEM((4, _CHUNK), jnp.int32),
            pltpu.VMEM((4, _CHUNK), jnp.int32),
            pltpu.VMEM((2, _CHUNK, _C), jnp.float32),
            pltpu.SemaphoreType.DMA((2,)),
            pltpu.SemaphoreType.DMA((2,)),
        ],
    )
    def _assemble(xflat_hbm, cls_hbm, src_hbm, dst_hbm, out_hbm,
                  src_v, dst_v, rows_v, gsem, ssem):
        cid = lax.axis_index("c")
        sid = lax.axis_index("s")
        w = sid * 2 + cid
        pltpu.sync_copy(src_hbm.at[w], src_v)
        pltpu.sync_copy(dst_hbm.at[w], dst_v)

        def gather(j):
            return pltpu.async_copy(xflat_hbm.at[src_v.at[j]],
                                    rows_v.at[j % 2], gsem.at[j % 2])

        def scatter(j):
            return pltpu.async_copy(rows_v.at[j % 2],
                                    out_hbm.at[sid].at[dst_v.at[j]],
                                    ssem.at[j % 2])

        gathers = {0: gather(0)}
        scatters = {}
        for j in range(4):
            gathers[j].wait()
            if j == 0:
                @pl.when(cid == 0)
                def _():
                    pltpu.sync_copy(cls_hbm.at[sid * 8], rows_v.at[0, 0])
            if j == 1:
                @pl.when(cid == 0)
                def _():
                    pltpu.sync_copy(cls_hbm.at[sid * 8 + 1], rows_v.at[1, 57])
            scatters[j] = scatter(j)
            if j + 1 < 4:
                if j >= 1:
                    scatters[j - 1].wait()
                gathers[j + 1] = gather(j + 1)
        scatters[2].wait()
        scatters[3].wait()

    return _assemble


def kernel(x, cls_tokens):
    b, c, t, h, w = x.shape
    n = h * w
    xt4 = jnp.transpose(x, (0, 3, 4, 2, 1)).reshape(b, n, t, c)
    xflat = xt4.reshape(b * n * t, c)
    cls_flat = cls_tokens.reshape(b * t, c)

    p = pl.pallas_call(
        _scores_kernel,
        grid=(b,),
        in_specs=[
            pl.BlockSpec(memory_space=pl.ANY),
            pl.BlockSpec((1, t, c), lambda i: (i, 0, 0)),
        ],
        out_specs=pl.BlockSpec((1, 1, n), lambda i: (i, 0, 0)),
        out_shape=jax.ShapeDtypeStruct((b, 1, n), jnp.float32),
        scratch_shapes=[
            pltpu.VMEM((2, n, c), jnp.float32),
            pltpu.SemaphoreType.DMA((2,)),
        ],
    )(xt4, cls_tokens)

    p_col = p.reshape(b, n, 1)
    sel = pl.pallas_call(
        _topk_kernel,
        grid=(b // _TB,),
        in_specs=[
            pl.BlockSpec((b, 1, n), lambda i: (0, 0, 0)),
            pl.BlockSpec((_TB, 1, n), lambda i: (i, 0, 0)),
            pl.BlockSpec((_TB, n, 1), lambda i: (i, 0, 0)),
        ],
        out_specs=pl.BlockSpec((_TB, 128, 1), lambda i: (i, 0, 0)),
        out_shape=jax.ShapeDtypeStruct((b, 128, 1), jnp.int32),
    )(p, p, p_col)
    sel_ids = sel[:, :_K, 0]

    batch_base = (jnp.arange(b, dtype=jnp.int32) * (n * t))[:, None]
    sel_rows = batch_base + sel_ids * t
    glob_rows = batch_base + jnp.arange(n, dtype=jnp.int32)[None, :] * t + 1
    zero = jnp.zeros((b, 1), jnp.int32)
    row_map = jnp.concatenate(
        [batch_base + zero, sel_rows, batch_base + zero, glob_rows], axis=1)
    row_map = row_map.reshape(_NW, _HALF)

    j_idx = jnp.minimum(
        jnp.arange(4, dtype=jnp.int32)[:, None] * _CHUNK
        + jnp.arange(_CHUNK, dtype=jnp.int32)[None, :],
        _HALF - 1)
    src_map = jnp.take_along_axis(
        row_map[:, None, :], j_idx[None], axis=2)
    dst_map = ((jnp.arange(_NW, dtype=jnp.int32) % 2) * _HALF)[:, None, None] \
        + j_idx[None]

    return _make_assemble()(xflat, cls_flat, src_map, dst_map)
